# baseline (speedup 1.0000x reference)
import functools
import math

import jax
import jax.numpy as jnp
from jax import lax
from jax.experimental import pallas as pl
from jax.experimental.pallas import tpu as pltpu
from jax.experimental.pallas import tpu_sc as plsc

D_MODEL = 512
LANES = 16
NUM_CORES = 2
NUM_SUBCORES = 16
NUM_WORKERS = NUM_CORES * NUM_SUBCORES
CHUNK = 32
NBUF = 4


@functools.lru_cache(maxsize=None)
def _make_embedding_kernel(batch: int):
    rows_per_worker = batch // NUM_WORKERS
    num_chunks = rows_per_worker // CHUNK
    assert num_chunks % NBUF == 0 and num_chunks >= 2 * NBUF
    scale = math.sqrt(D_MODEL)
    mesh = plsc.VectorSubcoreMesh(core_axis_name="c", subcore_axis_name="s")

    scratch = [pltpu.VMEM((rows_per_worker,), jnp.int32)]
    scratch += [pltpu.VMEM((CHUNK, D_MODEL), jnp.float32)] * NBUF
    scratch += [pltpu.SemaphoreType.DMA] * (2 * NBUF)

    @functools.partial(
        pl.kernel,
        out_type=jax.ShapeDtypeStruct((batch, D_MODEL), jnp.float32),
        mesh=mesh,
        scratch_types=scratch,
    )
    def emb_kernel(table_hbm, idx_hbm, out_hbm, idx_v, *rest):
        bufs = rest[:NBUF]
        gsems = rest[NBUF : 2 * NBUF]
        wsems = rest[2 * NBUF :]

        wid = lax.axis_index("s") * NUM_CORES + lax.axis_index("c")
        base = wid * rows_per_worker
        pltpu.sync_copy(idx_hbm.at[pl.ds(base, rows_per_worker)], idx_v)

        def start_gather(c, b):
            pltpu.async_copy(
                table_hbm.at[idx_v.at[pl.ds(c * CHUNK, CHUNK)]],
                bufs[b],
                gsems[b],
            )

        def wait_gather(c, b):
            pltpu.make_async_copy(
                table_hbm.at[idx_v.at[pl.ds(c * CHUNK, CHUNK)]],
                bufs[b],
                gsems[b],
            ).wait()

        def start_wb(c, b):
            pltpu.async_copy(
                bufs[b], out_hbm.at[pl.ds(base + c * CHUNK, CHUNK)], wsems[b]
            )

        def wait_wb(c, b):
            pltpu.make_async_copy(
                bufs[b], out_hbm.at[pl.ds(base + c * CHUNK, CHUNK)], wsems[b]
            ).wait()

        def scale_buf(b):
            buf = bufs[b]

            @pl.loop(0, CHUNK, unroll=4)
            def _(r):
                for c0 in range(0, D_MODEL, LANES):
                    buf[r, pl.ds(c0, LANES)] = buf[r, pl.ds(c0, LANES)] * scale

        def step(c, b, wb_wait, issue):
            wait_gather(c, b)
            scale_buf(b)
            start_wb(c, b)
            b2 = (b + 2) % NBUF
            if wb_wait:
                wait_wb(c - 2, b2)
            if issue:
                start_gather(c + 2, b2)

        start_gather(0, 0)
        start_gather(1, 1)
        step(0, 0, wb_wait=False, issue=True)
        step(1, 1, wb_wait=False, issue=True)
        step(2, 2, wb_wait=True, issue=True)
        step(3, 3, wb_wait=True, issue=True)

        @pl.loop(NBUF, num_chunks - NBUF, step=NBUF)
        def _(g):
            for b in range(NBUF):
                step(g + b, b, wb_wait=True, issue=True)

        c0 = num_chunks - NBUF
        step(c0 + 0, 0, wb_wait=True, issue=True)
        step(c0 + 1, 1, wb_wait=True, issue=True)
        step(c0 + 2, 2, wb_wait=True, issue=False)
        step(c0 + 3, 3, wb_wait=True, issue=False)
        wait_wb(c0 + 2, 2)
        wait_wb(c0 + 3, 3)

    return emb_kernel


@jax.jit
def kernel(x, table):
    b, s = x.shape
    idx = x.reshape(b * s).astype(jnp.int32)
    out = _make_embedding_kernel(b * s)(table, idx)
    return out.reshape(b, s, D_MODEL)


# speedup vs baseline: 1.6796x; 1.6796x over previous
import functools
import math

import jax
import jax.numpy as jnp
from jax import lax
from jax.experimental import pallas as pl
from jax.experimental.pallas import tpu as pltpu
from jax.experimental.pallas import tpu_sc as plsc

D_MODEL = 512
LANES = 16
NUM_CORES = 2
NUM_SUBCORES = 16
NUM_WORKERS = NUM_CORES * NUM_SUBCORES
CHUNK = 32
NBUF = 4


@functools.lru_cache(maxsize=None)
def _make_embedding_kernel(batch: int):
    rows_per_worker = batch // NUM_WORKERS
    num_chunks = rows_per_worker // CHUNK
    assert num_chunks % NBUF == 0 and num_chunks >= 2 * NBUF
    scale = math.sqrt(D_MODEL)
    mesh = plsc.VectorSubcoreMesh(core_axis_name="c", subcore_axis_name="s")

    scratch = [pltpu.VMEM((rows_per_worker,), jnp.int32)]
    scratch += [pltpu.VMEM((CHUNK, D_MODEL), jnp.float32)] * NBUF
    scratch += [pltpu.SemaphoreType.DMA] * (2 * NBUF)

    @functools.partial(
        pl.kernel,
        out_type=jax.ShapeDtypeStruct((batch, D_MODEL), jnp.float32),
        mesh=mesh,
        scratch_types=scratch,
    )
    def emb_kernel(table_hbm, idx_hbm, out_hbm, idx_v, *rest):
        bufs = rest[:NBUF]
        gsems = rest[NBUF : 2 * NBUF]
        wsems = rest[2 * NBUF :]

        wid = lax.axis_index("s") * NUM_CORES + lax.axis_index("c")
        base = wid * rows_per_worker
        pltpu.sync_copy(idx_hbm.at[pl.ds(base, rows_per_worker)], idx_v)

        def start_gather(c, b):
            pltpu.async_copy(
                table_hbm.at[idx_v.at[pl.ds(c * CHUNK, CHUNK)]],
                bufs[b],
                gsems[b],
            )

        def wait_gather(c, b):
            pltpu.make_async_copy(
                table_hbm.at[idx_v.at[pl.ds(c * CHUNK, CHUNK)]],
                bufs[b],
                gsems[b],
            ).wait()

        def start_wb(c, b):
            pltpu.async_copy(
                bufs[b], out_hbm.at[pl.ds(base + c * CHUNK, CHUNK)], wsems[b]
            )

        def wait_wb(c, b):
            pltpu.make_async_copy(
                bufs[b], out_hbm.at[pl.ds(base + c * CHUNK, CHUNK)], wsems[b]
            ).wait()

        def scale_buf(b):
            buf = bufs[b]

            @pl.loop(0, CHUNK)
            def _(r):
                for c0 in range(0, D_MODEL, LANES):
                    buf[r, pl.ds(c0, LANES)] = buf[r, pl.ds(c0, LANES)] * scale

        def step(c, b, wb_wait, issue):
            wait_gather(c, b)
            scale_buf(b)
            start_wb(c, b)
            b2 = (b + 2) % NBUF
            if wb_wait:
                wait_wb(c - 2, b2)
            if issue:
                start_gather(c + 2, b2)

        start_gather(0, 0)
        start_gather(1, 1)
        step(0, 0, wb_wait=False, issue=True)
        step(1, 1, wb_wait=False, issue=True)
        step(2, 2, wb_wait=True, issue=True)
        step(3, 3, wb_wait=True, issue=True)

        @pl.loop(NBUF, num_chunks - NBUF, step=NBUF)
        def _(g):
            for b in range(NBUF):
                step(g + b, b, wb_wait=True, issue=True)

        c0 = num_chunks - NBUF
        step(c0 + 0, 0, wb_wait=True, issue=True)
        step(c0 + 1, 1, wb_wait=True, issue=True)
        step(c0 + 2, 2, wb_wait=True, issue=False)
        step(c0 + 3, 3, wb_wait=True, issue=False)
        wait_wb(c0 + 2, 2)
        wait_wb(c0 + 3, 3)

    return emb_kernel


@jax.jit
def kernel(x, table):
    b, s = x.shape
    idx = x.reshape(b * s).astype(jnp.int32)
    out = _make_embedding_kernel(b * s)(table, idx)
    return out.reshape(b, s, D_MODEL)
